# baseline (speedup 1.0000x reference)
import functools

import jax
import jax.numpy as jnp
import numpy as np
from jax import lax
from jax.experimental import pallas as pl
from jax.experimental.pallas import tpu as pltpu
from jax.experimental.pallas import tpu_sc as plsc

_NV, _NE, _E, _D = 10000, 5000, 320000, 128
_NC, _NS, _L = 2, 16, 16
_NW = _NC * _NS
_B = 80


def _dense_in_body(v_ref, w_ref, b_ref, nw_ref, x_ref):
    x = lax.dot_general(v_ref[...], w_ref[...],
                        (((1,), (1,)), ((), ())),
                        preferred_element_type=jnp.float32)
    x_ref[...] = jnp.maximum(x + b_ref[...], 0.0) * nw_ref[...]


def _dense_in(v, W, b2, nw, blk):
    n = v.shape[0]
    grid = n // blk
    return pl.pallas_call(
        _dense_in_body,
        grid=(grid,),
        in_specs=[
            pl.BlockSpec((blk, _D), lambda i: (i, 0)),
            pl.BlockSpec((_D, _D), lambda i: (0, 0)),
            pl.BlockSpec((1, _D), lambda i: (0, 0)),
            pl.BlockSpec((blk, 1), lambda i: (i, 0)),
        ],
        out_specs=pl.BlockSpec((blk, _D), lambda i: (i, 0)),
        out_shape=jax.ShapeDtypeStruct((n, _D), jnp.float32),
    )(v, W, b2, nw)


def _dense_mid_body(e_ref, s0_ref, s1_ref, ers_ref, w_ref, b_ref, ew_ref,
                    e1_ref, y_ref):
    e1 = (e_ref[...] + s0_ref[...] + s1_ref[...]) / ers_ref[...]
    e1_ref[...] = e1
    y = lax.dot_general(e1, w_ref[...], (((1,), (1,)), ((), ())),
                        preferred_element_type=jnp.float32)
    y_ref[...] = jnp.maximum(y + b_ref[...], 0.0) * ew_ref[...]


def _dense_mid(e, s0, s1, ers, W, b2, ew, blk):
    n = e.shape[0]
    grid = n // blk
    row = lambda i: (i, 0)
    fixed = lambda i: (0, 0)
    return pl.pallas_call(
        _dense_mid_body,
        grid=(grid,),
        in_specs=[
            pl.BlockSpec((blk, _D), row),
            pl.BlockSpec((blk, _D), row),
            pl.BlockSpec((blk, _D), row),
            pl.BlockSpec((blk, 1), row),
            pl.BlockSpec((_D, _D), fixed),
            pl.BlockSpec((1, _D), fixed),
            pl.BlockSpec((blk, 1), row),
        ],
        out_specs=[pl.BlockSpec((blk, _D), row), pl.BlockSpec((blk, _D), row)],
        out_shape=[jax.ShapeDtypeStruct((n, _D), jnp.float32),
                   jax.ShapeDtypeStruct((n, _D), jnp.float32)],
    )(e, s0, s1, ers, W, b2, ew)


def _combine_body(v_ref, nw_ref, t0_ref, t1_ref, nrs_ref, out_ref):
    out_ref[...] = (v_ref[...] * nw_ref[...] + t0_ref[...] + t1_ref[...]) \
        / nrs_ref[...]


def _combine(v, nw, t0, t1, nrs, blk):
    n = v.shape[0]
    grid = n // blk
    row = lambda i: (i, 0)
    return pl.pallas_call(
        _combine_body,
        grid=(grid,),
        in_specs=[
            pl.BlockSpec((blk, _D), row),
            pl.BlockSpec((blk, 1), row),
            pl.BlockSpec((blk, _D), row),
            pl.BlockSpec((blk, _D), row),
            pl.BlockSpec((blk, 1), row),
        ],
        out_specs=pl.BlockSpec((blk, _D), row),
        out_shape=jax.ShapeDtypeStruct((n, _D), jnp.float32),
    )(v, nw, t0, t1, nrs)


def _make_scatter(n_dst):
    per_w = _E // _NW
    n_chunks = per_w // _B
    rps = (n_dst // 8 // _NS) * 8
    tail = n_dst - _NS * rps
    mesh = plsc.VectorSubcoreMesh(core_axis_name="c", subcore_axis_name="s")

    @functools.partial(
        pl.kernel, mesh=mesh,
        compiler_params=pltpu.CompilerParams(needs_layout_passes=False),
        out_type=jax.ShapeDtypeStruct((_NC, n_dst, _D), jnp.float32),
        scratch_types=[
            pltpu.VMEM((3, _B, _D), jnp.float32),
            pltpu.VMEM((_B,), jnp.float32),
            pltpu.VMEM((_B,), jnp.float32),
            pltpu.VMEM((_B,), jnp.float32),
            pltpu.VMEM((_B,), jnp.int32),
            pltpu.VMEM((_B,), jnp.int32),
            pltpu.VMEM((_B,), jnp.int32),
            pltpu.VMEM((_B,), jnp.int32),
            pltpu.VMEM((_B,), jnp.int32),
            pltpu.VMEM((_B,), jnp.int32),
            pltpu.VMEM_SHARED((n_dst, _D), jnp.float32),
            pltpu.SemaphoreType.DMA,
            pltpu.SemaphoreType.DMA,
            pltpu.SemaphoreType.DMA,
            pltpu.SemaphoreType.DMA,
            pltpu.SemaphoreType.DMA,
            pltpu.SemaphoreType.DMA,
            pltpu.SemaphoreType.DMA,
            pltpu.SemaphoreType.DMA,
            pltpu.SemaphoreType.DMA,
        ],
    )
    def k(x_hbm, si_hbm, di_hbm, w_hbm, out_hbm, rows_v,
          wv0, wv1, wv2, si0, si1, si2, di0, di1, di2,
          acc_sh, rs0, rs1, rs2, gs0, gs1, gs2, ss0, ss1, ss2):
        c = lax.axis_index("c")
        s = lax.axis_index("s")
        wid = s * _NC + c
        wvs = (wv0, wv1, wv2)
        sis = (si0, si1, si2)
        dis = (di0, di1, di2)
        rse = (rs0, rs1, rs2)
        gse = (gs0, gs1, gs2)
        sse = (ss0, ss1, ss2)
        zero16 = jnp.zeros((_L,), jnp.float32)

        def rec_descs(t, b):
            base = wid * per_w + t * _B
            return (
                pltpu.make_async_copy(
                    si_hbm.at[pl.ds(base, _B)], sis[b], rse[b]),
                pltpu.make_async_copy(
                    di_hbm.at[pl.ds(base, _B)], dis[b], rse[b]),
                pltpu.make_async_copy(
                    w_hbm.at[pl.ds(base, _B)], wvs[b], rse[b]),
            )

        def start_rec(t, b):
            for d in rec_descs(t, b):
                d.start()

        def wait_rec(t, b):
            for d in rec_descs(t, b):
                d.wait()

        def start_gather(t, b):
            wait_rec(t, b)
            pltpu.async_copy(x_hbm.at[sis[b]], rows_v.at[b], gse[b])

        def wait_gather(b):
            pltpu.make_async_copy(
                x_hbm.at[sis[b]], rows_v.at[b], gse[b]).wait()

        def start_scatter(b):
            pltpu.async_copy(rows_v.at[b], acc_sh.at[dis[b]], sse[b],
                             add=True)

        def wait_scatter(b):
            pltpu.make_async_copy(
                rows_v.at[b], acc_sh.at[dis[b]], sse[b]).wait()

        def scale(b):
            def scale_g(g, carry2):
                wg = wvs[b][pl.ds(g * _L, _L)]
                for l in range(_L):
                    wv = jnp.full((_L,), wg[l], jnp.float32)
                    r = g * _L + l
                    for j in range(_D // _L):
                        sl = pl.ds(j * _L, _L)
                        rows_v[b, r, sl] = rows_v[b, r, sl] * wv
                return carry2
            lax.fori_loop(0, _B // _L, scale_g, 0)

        start_rec(0, 0)
        start_rec(1, 1)
        start_rec(2, 2)

        def zero_rows(r, carry):
            for j in range(_D // _L):
                rows_v[0, r, pl.ds(j * _L, _L)] = zero16
            return carry
        lax.fori_loop(0, _B, zero_rows, 0)

        def zero_range(base, length):
            for t in range(0, length, _B):
                nrows = min(_B, length - t)
                pltpu.sync_copy(rows_v.at[0, pl.ds(0, nrows)],
                                acc_sh.at[pl.ds(base + t, nrows)])
        zero_range(s * rps, rps)

        @pl.when(s == 0)
        def _():
            zero_range(_NS * rps, tail)
        plsc.subcore_barrier()

        start_gather(0, 0)
        start_gather(1, 1)

        wait_gather(0)
        scale(0)
        start_scatter(0)
        start_gather(2, 2)
        start_rec(3, 0)

        n_main = (n_chunks - 5) // 3

        def triple(p, carry):
            for o in range(3):
                t = 1 + p * 3 + o
                b = (1 + o) % 3
                bp = o % 3
                wait_gather(b)
                scale(b)
                start_scatter(b)
                wait_scatter(bp)
                start_gather(t + 2, bp)
                start_rec(t + 3, b)
            return carry
        lax.fori_loop(0, n_main, triple, 0)

        for t in range(1 + 3 * n_main, n_chunks):
            b = t % 3
            wait_gather(b)
            scale(b)
            start_scatter(b)
            if t + 2 < n_chunks:
                bp = (t + 2) % 3
                wait_scatter(bp)
                start_gather(t + 2, bp)
            if t + 3 < n_chunks:
                start_rec(t + 3, b)
        for b in range(3):
            wait_scatter(b)

        plsc.subcore_barrier()
        pltpu.sync_copy(acc_sh.at[pl.ds(s * rps, rps)],
                        out_hbm.at[c, pl.ds(s * rps, rps)])

        @pl.when(s == 0)
        def _():
            pltpu.sync_copy(acc_sh.at[pl.ds(_NS * rps, tail)],
                            out_hbm.at[c, pl.ds(_NS * rps, tail)])

    return k


_scatter_e = _make_scatter(_NE)
_scatter_v = _make_scatter(_NV)


def kernel(v, e, W_v2e, b_v2e, W_e2v, b_e2v, n_weight, e_weight,
           n_reg_weight, e_reg_weight, e_reg_sum, n_reg_sum,
           vidx, eidx, ve_lists):
    ve0 = jnp.asarray(ve_lists[:, 0])
    ve1 = jnp.asarray(ve_lists[:, 1])
    w_e = n_reg_weight[:, 0]
    w_v2 = e_reg_weight[:, 0]

    x = _dense_in(v, W_v2e, b_v2e.reshape(1, _D), n_weight, 2000)
    s = _scatter_e(x, ve0, eidx, w_e)
    e1, y = _dense_mid(e, s[0], s[1], e_reg_sum,
                       W_e2v, b_e2v.reshape(1, _D), e_weight, 1000)
    t = _scatter_v(y, ve1, vidx, w_v2)
    v2 = _combine(v, n_weight, t[0], t[1], n_reg_sum, 2000)
    return (v2, e1)


# speedup vs baseline: 1.0364x; 1.0364x over previous
import functools

import jax
import jax.numpy as jnp
import numpy as np
from jax import lax
from jax.experimental import pallas as pl
from jax.experimental.pallas import tpu as pltpu
from jax.experimental.pallas import tpu_sc as plsc

_NV, _NE, _E, _D = 10000, 5000, 320000, 128
_NC, _NS, _L = 2, 16, 16
_NW = _NC * _NS
_B = 80


def _dense_in_body(v_ref, w_ref, b_ref, nw_ref, x_ref):
    x = lax.dot_general(v_ref[...], w_ref[...],
                        (((1,), (1,)), ((), ())),
                        preferred_element_type=jnp.float32)
    x_ref[...] = jnp.maximum(x + b_ref[...], 0.0) * nw_ref[...]


def _dense_in(v, W, b2, nw, blk):
    n = v.shape[0]
    grid = n // blk
    return pl.pallas_call(
        _dense_in_body,
        grid=(grid,),
        in_specs=[
            pl.BlockSpec((blk, _D), lambda i: (i, 0)),
            pl.BlockSpec((_D, _D), lambda i: (0, 0)),
            pl.BlockSpec((1, _D), lambda i: (0, 0)),
            pl.BlockSpec((blk, 1), lambda i: (i, 0)),
        ],
        out_specs=pl.BlockSpec((blk, _D), lambda i: (i, 0)),
        out_shape=jax.ShapeDtypeStruct((n, _D), jnp.float32),
    )(v, W, b2, nw)


def _dense_mid_body(e_ref, s0_ref, s1_ref, ers_ref, w_ref, b_ref, ew_ref,
                    e1_ref, y_ref):
    e1 = (e_ref[...] + s0_ref[...] + s1_ref[...]) / ers_ref[...]
    e1_ref[...] = e1
    y = lax.dot_general(e1, w_ref[...], (((1,), (1,)), ((), ())),
                        preferred_element_type=jnp.float32)
    y_ref[...] = jnp.maximum(y + b_ref[...], 0.0) * ew_ref[...]


def _dense_mid(e, s0, s1, ers, W, b2, ew, blk):
    n = e.shape[0]
    grid = n // blk
    row = lambda i: (i, 0)
    fixed = lambda i: (0, 0)
    return pl.pallas_call(
        _dense_mid_body,
        grid=(grid,),
        in_specs=[
            pl.BlockSpec((blk, _D), row),
            pl.BlockSpec((blk, _D), row),
            pl.BlockSpec((blk, _D), row),
            pl.BlockSpec((blk, 1), row),
            pl.BlockSpec((_D, _D), fixed),
            pl.BlockSpec((1, _D), fixed),
            pl.BlockSpec((blk, 1), row),
        ],
        out_specs=[pl.BlockSpec((blk, _D), row), pl.BlockSpec((blk, _D), row)],
        out_shape=[jax.ShapeDtypeStruct((n, _D), jnp.float32),
                   jax.ShapeDtypeStruct((n, _D), jnp.float32)],
    )(e, s0, s1, ers, W, b2, ew)


def _combine_body(v_ref, nw_ref, t0_ref, t1_ref, nrs_ref, out_ref):
    out_ref[...] = (v_ref[...] * nw_ref[...] + t0_ref[...] + t1_ref[...]) \
        / nrs_ref[...]


def _combine(v, nw, t0, t1, nrs, blk):
    n = v.shape[0]
    grid = n // blk
    row = lambda i: (i, 0)
    return pl.pallas_call(
        _combine_body,
        grid=(grid,),
        in_specs=[
            pl.BlockSpec((blk, _D), row),
            pl.BlockSpec((blk, 1), row),
            pl.BlockSpec((blk, _D), row),
            pl.BlockSpec((blk, _D), row),
            pl.BlockSpec((blk, 1), row),
        ],
        out_specs=pl.BlockSpec((blk, _D), row),
        out_shape=jax.ShapeDtypeStruct((n, _D), jnp.float32),
    )(v, nw, t0, t1, nrs)


def _make_scatter(n_dst):
    per_w = _E // _NW
    n_chunks = per_w // _B
    rps = (n_dst // 8 // _NS) * 8
    tail = n_dst - _NS * rps
    mesh = plsc.VectorSubcoreMesh(core_axis_name="c", subcore_axis_name="s")

    @functools.partial(
        pl.kernel, mesh=mesh,
        compiler_params=pltpu.CompilerParams(needs_layout_passes=False),
        out_type=[jax.ShapeDtypeStruct((n_dst, _D), jnp.float32),
                  jax.ShapeDtypeStruct((n_dst, _D), jnp.float32)],
        scratch_types=[
            pltpu.VMEM((3, _B, _D), jnp.float32),
            pltpu.VMEM((_B,), jnp.float32),
            pltpu.VMEM((_B,), jnp.float32),
            pltpu.VMEM((_B,), jnp.float32),
            pltpu.VMEM((_B,), jnp.int32),
            pltpu.VMEM((_B,), jnp.int32),
            pltpu.VMEM((_B,), jnp.int32),
            pltpu.VMEM((_B,), jnp.int32),
            pltpu.VMEM((_B,), jnp.int32),
            pltpu.VMEM((_B,), jnp.int32),
            pltpu.VMEM_SHARED((n_dst, _D), jnp.float32),
            pltpu.SemaphoreType.DMA,
            pltpu.SemaphoreType.DMA,
            pltpu.SemaphoreType.DMA,
            pltpu.SemaphoreType.DMA,
            pltpu.SemaphoreType.DMA,
            pltpu.SemaphoreType.DMA,
            pltpu.SemaphoreType.DMA,
            pltpu.SemaphoreType.DMA,
            pltpu.SemaphoreType.DMA,
        ],
    )
    def k(x_hbm, si_hbm, di_hbm, w_hbm, out0_hbm, out1_hbm, rows_v,
          wv0, wv1, wv2, si0, si1, si2, di0, di1, di2,
          acc_sh, rs0, rs1, rs2, gs0, gs1, gs2, ss0, ss1, ss2):
        c = lax.axis_index("c")
        s = lax.axis_index("s")
        wid = s * _NC + c
        wvs = (wv0, wv1, wv2)
        sis = (si0, si1, si2)
        dis = (di0, di1, di2)
        rse = (rs0, rs1, rs2)
        gse = (gs0, gs1, gs2)
        sse = (ss0, ss1, ss2)
        zero16 = jnp.zeros((_L,), jnp.float32)

        def rec_descs(t, b):
            base = wid * per_w + t * _B
            return (
                pltpu.make_async_copy(
                    si_hbm.at[pl.ds(base, _B)], sis[b], rse[b]),
                pltpu.make_async_copy(
                    di_hbm.at[pl.ds(base, _B)], dis[b], rse[b]),
                pltpu.make_async_copy(
                    w_hbm.at[pl.ds(base, _B)], wvs[b], rse[b]),
            )

        def start_rec(t, b):
            for d in rec_descs(t, b):
                d.start()

        def wait_rec(t, b):
            for d in rec_descs(t, b):
                d.wait()

        def start_gather(t, b):
            wait_rec(t, b)
            pltpu.async_copy(x_hbm.at[sis[b]], rows_v.at[b], gse[b])

        def wait_gather(b):
            pltpu.make_async_copy(
                x_hbm.at[sis[b]], rows_v.at[b], gse[b]).wait()

        def start_scatter(b):
            pltpu.async_copy(rows_v.at[b], acc_sh.at[dis[b]], sse[b],
                             add=True)

        def wait_scatter(b):
            pltpu.make_async_copy(
                rows_v.at[b], acc_sh.at[dis[b]], sse[b]).wait()

        def scale(b):
            def scale_g(g, carry2):
                wg = wvs[b][pl.ds(g * _L, _L)]
                for l in range(_L):
                    wv = jnp.full((_L,), wg[l], jnp.float32)
                    r = g * _L + l
                    for j in range(_D // _L):
                        sl = pl.ds(j * _L, _L)
                        rows_v[b, r, sl] = rows_v[b, r, sl] * wv
                return carry2
            lax.fori_loop(0, _B // _L, scale_g, 0)

        start_rec(0, 0)
        start_rec(1, 1)
        start_rec(2, 2)

        def zero_rows(r, carry):
            for j in range(_D // _L):
                rows_v[0, r, pl.ds(j * _L, _L)] = zero16
            return carry
        lax.fori_loop(0, _B, zero_rows, 0)

        def zero_range(base, length):
            for t in range(0, length, _B):
                nrows = min(_B, length - t)
                pltpu.sync_copy(rows_v.at[0, pl.ds(0, nrows)],
                                acc_sh.at[pl.ds(base + t, nrows)])
        zero_range(s * rps, rps)

        @pl.when(s == 0)
        def _():
            zero_range(_NS * rps, tail)
        plsc.subcore_barrier()

        start_gather(0, 0)
        start_gather(1, 1)

        wait_gather(0)
        scale(0)
        start_scatter(0)
        start_gather(2, 2)
        start_rec(3, 0)

        n_main = (n_chunks - 5) // 3

        def triple(p, carry):
            for o in range(3):
                t = 1 + p * 3 + o
                b = (1 + o) % 3
                bp = o % 3
                wait_gather(b)
                scale(b)
                start_scatter(b)
                wait_scatter(bp)
                start_gather(t + 2, bp)
                start_rec(t + 3, b)
            return carry
        lax.fori_loop(0, n_main, triple, 0)

        for t in range(1 + 3 * n_main, n_chunks):
            b = t % 3
            wait_gather(b)
            scale(b)
            start_scatter(b)
            if t + 2 < n_chunks:
                bp = (t + 2) % 3
                wait_scatter(bp)
                start_gather(t + 2, bp)
            if t + 3 < n_chunks:
                start_rec(t + 3, b)
        for b in range(3):
            wait_scatter(b)

        plsc.subcore_barrier()

        def writeout(out_hbm):
            pltpu.sync_copy(acc_sh.at[pl.ds(s * rps, rps)],
                            out_hbm.at[pl.ds(s * rps, rps)])

            @pl.when(s == 0)
            def _():
                pltpu.sync_copy(acc_sh.at[pl.ds(_NS * rps, tail)],
                                out_hbm.at[pl.ds(_NS * rps, tail)])

        @pl.when(c == 0)
        def _():
            writeout(out0_hbm)

        @pl.when(c == 1)
        def _():
            writeout(out1_hbm)

    return k


_scatter_e = _make_scatter(_NE)
_scatter_v = _make_scatter(_NV)


def kernel(v, e, W_v2e, b_v2e, W_e2v, b_e2v, n_weight, e_weight,
           n_reg_weight, e_reg_weight, e_reg_sum, n_reg_sum,
           vidx, eidx, ve_lists):
    ve0 = jnp.asarray(ve_lists[:, 0])
    ve1 = jnp.asarray(ve_lists[:, 1])
    w_e = n_reg_weight[:, 0]
    w_v2 = e_reg_weight[:, 0]

    x = _dense_in(v, W_v2e, b_v2e.reshape(1, _D), n_weight, 2000)
    s0, s1 = _scatter_e(x, ve0, eidx, w_e)
    e1, y = _dense_mid(e, s0, s1, e_reg_sum,
                       W_e2v, b_e2v.reshape(1, _D), e_weight, 1000)
    t0, t1 = _scatter_v(y, ve1, vidx, w_v2)
    v2 = _combine(v, n_weight, t0, t1, n_reg_sum, 2000)
    return (v2, e1)


# speedup vs baseline: 1.0374x; 1.0010x over previous
import functools

import jax
import jax.numpy as jnp
import numpy as np
from jax import lax
from jax.experimental import pallas as pl
from jax.experimental.pallas import tpu as pltpu
from jax.experimental.pallas import tpu_sc as plsc

_NV, _NE, _E, _D = 10000, 5000, 320000, 128
_NC, _NS, _L = 2, 16, 16
_NW = _NC * _NS
_B = 80


def _dense_in_body(v_ref, w_ref, b_ref, nw_ref, x_ref):
    x = lax.dot_general(v_ref[...], w_ref[...],
                        (((1,), (1,)), ((), ())),
                        preferred_element_type=jnp.float32)
    x_ref[...] = jnp.maximum(x + b_ref[...], 0.0) * nw_ref[...]


def _dense_in(v, W, b2, nw, blk):
    n = v.shape[0]
    grid = n // blk
    return pl.pallas_call(
        _dense_in_body,
        grid=(grid,),
        in_specs=[
            pl.BlockSpec((blk, _D), lambda i: (i, 0)),
            pl.BlockSpec((_D, _D), lambda i: (0, 0)),
            pl.BlockSpec((1, _D), lambda i: (0, 0)),
            pl.BlockSpec((blk, 1), lambda i: (i, 0)),
        ],
        out_specs=pl.BlockSpec((blk, _D), lambda i: (i, 0)),
        out_shape=jax.ShapeDtypeStruct((n, _D), jnp.float32),
    )(v, W, b2, nw)


def _dense_mid_body(e_ref, s0_ref, s1_ref, ers_ref, w_ref, b_ref, ew_ref,
                    e1_ref, y_ref):
    e1 = (e_ref[...] + s0_ref[...] + s1_ref[...]) / ers_ref[...]
    e1_ref[...] = e1
    y = lax.dot_general(e1, w_ref[...], (((1,), (1,)), ((), ())),
                        preferred_element_type=jnp.float32)
    y_ref[...] = jnp.maximum(y + b_ref[...], 0.0) * ew_ref[...]


def _dense_mid(e, s0, s1, ers, W, b2, ew, blk):
    n = e.shape[0]
    grid = n // blk
    row = lambda i: (i, 0)
    fixed = lambda i: (0, 0)
    return pl.pallas_call(
        _dense_mid_body,
        grid=(grid,),
        in_specs=[
            pl.BlockSpec((blk, _D), row),
            pl.BlockSpec((blk, _D), row),
            pl.BlockSpec((blk, _D), row),
            pl.BlockSpec((blk, 1), row),
            pl.BlockSpec((_D, _D), fixed),
            pl.BlockSpec((1, _D), fixed),
            pl.BlockSpec((blk, 1), row),
        ],
        out_specs=[pl.BlockSpec((blk, _D), row), pl.BlockSpec((blk, _D), row)],
        out_shape=[jax.ShapeDtypeStruct((n, _D), jnp.float32),
                   jax.ShapeDtypeStruct((n, _D), jnp.float32)],
    )(e, s0, s1, ers, W, b2, ew)


def _combine_body(v_ref, nw_ref, t0_ref, t1_ref, nrs_ref, out_ref):
    out_ref[...] = (v_ref[...] * nw_ref[...] + t0_ref[...] + t1_ref[...]) \
        / nrs_ref[...]


def _combine(v, nw, t0, t1, nrs, blk):
    n = v.shape[0]
    grid = n // blk
    row = lambda i: (i, 0)
    return pl.pallas_call(
        _combine_body,
        grid=(grid,),
        in_specs=[
            pl.BlockSpec((blk, _D), row),
            pl.BlockSpec((blk, 1), row),
            pl.BlockSpec((blk, _D), row),
            pl.BlockSpec((blk, _D), row),
            pl.BlockSpec((blk, 1), row),
        ],
        out_specs=pl.BlockSpec((blk, _D), row),
        out_shape=jax.ShapeDtypeStruct((n, _D), jnp.float32),
    )(v, nw, t0, t1, nrs)


def _make_scatter(n_dst):
    per_w = _E // _NW
    n_chunks = per_w // _B
    rps = (n_dst // 8 // _NS) * 8
    tail = n_dst - _NS * rps
    mesh = plsc.VectorSubcoreMesh(core_axis_name="c", subcore_axis_name="s")

    @functools.partial(
        pl.kernel, mesh=mesh,
        compiler_params=pltpu.CompilerParams(needs_layout_passes=False),
        out_type=[jax.ShapeDtypeStruct((n_dst, _D), jnp.float32),
                  jax.ShapeDtypeStruct((n_dst, _D), jnp.float32)],
        scratch_types=[
            pltpu.VMEM((3, _B, _D), jnp.float32),
            pltpu.VMEM((_B,), jnp.float32),
            pltpu.VMEM((_B,), jnp.float32),
            pltpu.VMEM((_B,), jnp.float32),
            pltpu.VMEM((_B,), jnp.int32),
            pltpu.VMEM((_B,), jnp.int32),
            pltpu.VMEM((_B,), jnp.int32),
            pltpu.VMEM((_B,), jnp.int32),
            pltpu.VMEM((_B,), jnp.int32),
            pltpu.VMEM((_B,), jnp.int32),
            pltpu.VMEM_SHARED((n_dst, _D), jnp.float32),
            pltpu.SemaphoreType.DMA,
            pltpu.SemaphoreType.DMA,
            pltpu.SemaphoreType.DMA,
            pltpu.SemaphoreType.DMA,
            pltpu.SemaphoreType.DMA,
            pltpu.SemaphoreType.DMA,
            pltpu.SemaphoreType.DMA,
            pltpu.SemaphoreType.DMA,
            pltpu.SemaphoreType.DMA,
        ],
    )
    def k(x_hbm, si_hbm, di_hbm, w_hbm, out0_hbm, out1_hbm, rows_v,
          wv0, wv1, wv2, si0, si1, si2, di0, di1, di2,
          acc_sh, rs0, rs1, rs2, gs0, gs1, gs2, ss0, ss1, ss2):
        c = lax.axis_index("c")
        s = lax.axis_index("s")
        wid = s * _NC + c
        wvs = (wv0, wv1, wv2)
        sis = (si0, si1, si2)
        dis = (di0, di1, di2)
        rse = (rs0, rs1, rs2)
        gse = (gs0, gs1, gs2)
        sse = (ss0, ss1, ss2)
        zero16 = jnp.zeros((_L,), jnp.float32)

        def rec_descs(t, b):
            base = wid * per_w + t * _B
            return (
                pltpu.make_async_copy(
                    si_hbm.at[pl.ds(base, _B)], sis[b], rse[b]),
                pltpu.make_async_copy(
                    di_hbm.at[pl.ds(base, _B)], dis[b], rse[b]),
                pltpu.make_async_copy(
                    w_hbm.at[pl.ds(base, _B)], wvs[b], rse[b]),
            )

        def start_rec(t, b):
            for d in rec_descs(t, b):
                d.start()

        def wait_rec(t, b):
            for d in rec_descs(t, b):
                d.wait()

        def start_gather(t, b):
            wait_rec(t, b)
            pltpu.async_copy(x_hbm.at[sis[b]], rows_v.at[b], gse[b])

        def wait_gather(b):
            pltpu.make_async_copy(
                x_hbm.at[sis[b]], rows_v.at[b], gse[b]).wait()

        def start_scatter(b):
            pltpu.async_copy(rows_v.at[b], acc_sh.at[dis[b]], sse[b],
                             add=True)

        def wait_scatter(b):
            pltpu.make_async_copy(
                rows_v.at[b], acc_sh.at[dis[b]], sse[b]).wait()

        def scale(b):
            def scale_g(g, carry2):
                wg = wvs[b][pl.ds(g * _L, _L)]
                for l in range(_L):
                    wv = jnp.full((_L,), wg[l], jnp.float32)
                    r = g * _L + l
                    for j in range(_D // _L):
                        sl = pl.ds(j * _L, _L)
                        rows_v[b, r, sl] = rows_v[b, r, sl] * wv
                return carry2
            lax.fori_loop(0, _B // _L, scale_g, 0)

        start_rec(0, 0)
        start_rec(1, 1)
        start_rec(2, 2)

        def zero_rows(r, carry):
            for j in range(_D // _L):
                rows_v[0, r, pl.ds(j * _L, _L)] = zero16
            return carry
        lax.fori_loop(0, _B, zero_rows, 0)

        def zero_range(base, length):
            for t in range(0, length, _B):
                nrows = min(_B, length - t)
                pltpu.sync_copy(rows_v.at[0, pl.ds(0, nrows)],
                                acc_sh.at[pl.ds(base + t, nrows)])
        zero_range(s * rps, rps)

        @pl.when(s == 0)
        def _():
            zero_range(_NS * rps, tail)
        plsc.subcore_barrier()

        start_gather(0, 0)
        start_gather(1, 1)

        wait_gather(0)
        scale(0)
        start_scatter(0)
        start_gather(2, 2)
        start_rec(3, 0)

        n_main = (n_chunks - 5) // 3

        def triple(p, carry):
            for o in range(3):
                t = 1 + p * 3 + o
                b = (1 + o) % 3
                bp = o % 3
                wait_gather(b)
                scale(b)
                start_scatter(b)
                wait_scatter(bp)
                start_gather(t + 2, bp)
                start_rec(t + 3, b)
            return carry
        lax.fori_loop(0, n_main, triple, 0)

        for t in range(1 + 3 * n_main, n_chunks):
            b = t % 3
            wait_gather(b)
            scale(b)
            start_scatter(b)
            if t + 2 < n_chunks:
                bp = (t + 2) % 3
                wait_scatter(bp)
                start_gather(t + 2, bp)
            if t + 3 < n_chunks:
                start_rec(t + 3, b)
        for b in range(3):
            wait_scatter(b)

        plsc.subcore_barrier()

        def writeout(out_hbm):
            pltpu.sync_copy(acc_sh.at[pl.ds(s * rps, rps)],
                            out_hbm.at[pl.ds(s * rps, rps)])

            @pl.when(s == 0)
            def _():
                pltpu.sync_copy(acc_sh.at[pl.ds(_NS * rps, tail)],
                                out_hbm.at[pl.ds(_NS * rps, tail)])

        @pl.when(c == 0)
        def _():
            writeout(out0_hbm)

        @pl.when(c == 1)
        def _():
            writeout(out1_hbm)

    return k


_scatter_e = _make_scatter(_NE)
_scatter_v = _make_scatter(_NV)


def kernel(v, e, W_v2e, b_v2e, W_e2v, b_e2v, n_weight, e_weight,
           n_reg_weight, e_reg_weight, e_reg_sum, n_reg_sum,
           vidx, eidx, ve_lists):
    ve0 = jnp.asarray(ve_lists[:, 0])
    ve1 = jnp.asarray(ve_lists[:, 1])
    w_e = n_reg_weight[:, 0]
    w_v2 = e_reg_weight[:, 0]

    x = _dense_in(v, W_v2e, b_v2e.reshape(1, _D), n_weight, 10000)
    s0, s1 = _scatter_e(x, ve0, eidx, w_e)
    e1, y = _dense_mid(e, s0, s1, e_reg_sum,
                       W_e2v, b_e2v.reshape(1, _D), e_weight, 5000)
    t0, t1 = _scatter_v(y, ve1, vidx, w_v2)
    v2 = _combine(v, n_weight, t0, t1, n_reg_sum, 10000)
    return (v2, e1)
